# baseline (speedup 1.0000x reference)
import functools

import jax
import jax.numpy as jnp
from jax import lax
from jax.experimental import pallas as pl
from jax.experimental.pallas import tpu as pltpu
from jax.experimental.pallas import tpu_sc as plsc

NC = 2
NS = 16
L = 16
NW = NC * NS

BATCH = 16384
D = 64
BPW = BATCH // NW
GROUPS = BPW // L

_mesh = plsc.VectorSubcoreMesh(core_axis_name="c", subcore_axis_name="s")


@functools.partial(
    pl.kernel,
    out_type=jax.ShapeDtypeStruct((BATCH,), jnp.float32),
    mesh=_mesh,
    scratch_types=[
        pltpu.VMEM((BPW,), jnp.int32),
        pltpu.VMEM((BPW,), jnp.int32),
        pltpu.VMEM((BPW, D), jnp.float32),
        pltpu.VMEM((BPW, D), jnp.float32),
        pltpu.VMEM((BPW,), jnp.float32),
        pltpu.SemaphoreType.DMA,
        pltpu.SemaphoreType.DMA,
    ],
    compiler_params=pltpu.CompilerParams(
        needs_layout_passes=False, use_tc_tiling_on_sc=False),
)
def _two_tower_sc(user_ids, banner_ids, user_table, banner_table, out_hbm,
                  uid_v, bid_v, urows_v, brows_v, out_v, sem_u, sem_b):
    wid = lax.axis_index("s") * NC + lax.axis_index("c")
    base = wid * BPW

    pltpu.sync_copy(user_ids.at[pl.ds(base, BPW)], uid_v)
    pltpu.sync_copy(banner_ids.at[pl.ds(base, BPW)], bid_v)
    cu = pltpu.async_copy(user_table.at[uid_v], urows_v, sem_u)
    cb = pltpu.async_copy(banner_table.at[bid_v], brows_v, sem_b)
    cu.wait()
    cb.wait()

    lanes = lax.iota(jnp.int32, L)

    @pl.loop(0, GROUPS)
    def _group(g):
        acc = jnp.zeros((L,), jnp.float32)
        for e in range(L):
            row = g * L + e
            s = jnp.zeros((L,), jnp.float32)
            for c in range(D // L):
                pu = urows_v[row, pl.ds(c * L, L)]
                pb = brows_v[row, pl.ds(c * L, L)]
                s = s + pu * pb
            acc = jnp.where(lanes == e, jnp.sum(s), acc)
        out_v[pl.ds(g * L, L)] = acc

    pltpu.sync_copy(out_v, out_hbm.at[pl.ds(base, BPW)])


def kernel(user_ids, banner_ids, user_table, banner_table):
    return _two_tower_sc(user_ids, banner_ids, user_table, banner_table)
